# baseline (speedup 1.0000x reference)
import functools

import jax
import jax.numpy as jnp
from jax import lax
from jax.experimental import pallas as pl
from jax.experimental.pallas import tpu as pltpu
from jax.experimental.pallas import tpu_sc as plsc

NUM_FEATURES = 26
VOCAB = 100000
EMBED_DIM = 32
BATCH = 16384
N = BATCH * NUM_FEATURES
L = 16

SEG = 2048
NSEG = BATCH // SEG
WIDE = 3200
NARROW = 3072
TAIL = 32
SPLIT = 13 * WIDE
CHUNK = 64
SBASE = SEG // 2 + L
GPC = CHUNK // L


def _make_sc_gather():
  mesh = plsc.VectorSubcoreMesh(core_axis_name="c", subcore_axis_name="s")

  @functools.partial(
      pl.kernel,
      mesh=mesh,
      compiler_params=pltpu.CompilerParams(
          use_tc_tiling_on_sc=True, needs_layout_passes=False),
      out_type=jax.ShapeDtypeStruct((N + CHUNK, 128), jnp.float32),
      scratch_types=[
          pltpu.VMEM((32, WIDE), jnp.float32),
          pltpu.VMEM((SEG,), jnp.int32),
          pltpu.VMEM((SEG + 2 * L,), jnp.int32),
          pltpu.VMEM((SEG + 2 * L,), jnp.int32),
          pltpu.VMEM((CHUNK, 128), jnp.float32),
          pltpu.VMEM((CHUNK, 128), jnp.float32),
          pltpu.VMEM((CHUNK,), jnp.int32),
          pltpu.VMEM((CHUNK,), jnp.int32),
          pltpu.VMEM((32, TAIL), jnp.float32),
          pltpu.SemaphoreType.DMA,
          pltpu.SemaphoreType.DMA,
          pltpu.SemaphoreType.DMA,
      ],
  )
  def sc_gather(table_hbm, feat_hbm, out_hbm, stage_v, featseg_v, cv_v, cj_v,
                outst0_v, outst1_v, sidx0_v, sidx1_v, tail_v, sem_stage,
                sem_s0, sem_s1):
    info = plsc.get_sparse_core_info()
    nc = info.num_cores
    wid = lax.axis_index("s") * nc + lax.axis_index("c")

    lo = jnp.where(wid < 13, wid * WIDE, SPLIT + (wid - 13) * NARROW)
    hi = lo + jnp.where(wid < 13, WIDE, NARROW) + jnp.where(
        wid == 31, TAIL, 0)

    iota = lax.iota(jnp.int32, L)
    iota26 = iota * NUM_FEATURES

    outst = (outst0_v, outst1_v)
    sidx = (sidx0_v, sidx1_v)
    sems = (sem_s0, sem_s1)

    def scat_copy(p):
      return pltpu.make_async_copy(outst[p], out_hbm.at[sidx[p]], sems[p])

    def fill_chunk(base, c, cnt, p):
      for g in range(GPC):
        off = base + c * CHUNK + g * L
        valid = (off - base + iota) < cnt
        vloc = cv_v[pl.ds(off, L)]
        vloc = jnp.where(valid, vloc, 0)
        j = cj_v[pl.ds(off, L)]
        j = jnp.where(valid, j, N + g * L + iota)
        sidx[p][pl.ds(g * L, L)] = j
        rows = lax.iota(jnp.int32, L) + g * L
        for e0 in range(0, EMBED_DIM, 8):
          vals = [
              plsc.load_gather(
                  stage_v, [jnp.full((L,), e0 + i, jnp.int32), vloc])
              for i in range(8)
          ]
          for i in range(8):
            plsc.store_scatter(
                outst[p], [rows, jnp.full((L,), e0 + i, jnp.int32)], vals[i])
      scat_copy(p).start()

    def per_feature(f, carry):
      fired0, fired1, gc = carry

      cp_wide = pltpu.make_async_copy(
          table_hbm.at[f, :, pl.ds(lo, WIDE)], stage_v, sem_stage)
      cp_narrow = pltpu.make_async_copy(
          table_hbm.at[f, :, pl.ds(lo, NARROW)],
          stage_v.at[:, pl.ds(0, NARROW)], sem_stage)
      cp_tail = pltpu.make_async_copy(
          table_hbm.at[f, :, pl.ds(VOCAB - TAIL, TAIL)], tail_v, sem_stage)

      @pl.when(wid < 13)
      def _():
        cp_wide.start()

      @pl.when(wid >= 13)
      def _():
        cp_narrow.start()

      @pl.when(wid == 31)
      def _():
        cp_tail.start()

      def per_segment(s, carry):
        fired0, fired1, gc = carry
        pltpu.sync_copy(feat_hbm.at[f, pl.ds(s * SEG, SEG)], featseg_v)

        def scan_it(k, carry):
          ptra, ptrb = carry
          va = featseg_v[pl.ds(k * 2 * L, L)]
          vb = featseg_v[pl.ds(k * 2 * L + L, L)]
          ma = (va >= lo) & (va < hi)
          mb = (vb >= lo) & (vb < hi)
          ja = iota26 + ((s * SEG + k * 2 * L) * NUM_FEATURES + f)
          jb = ja + L * NUM_FEATURES
          plsc.store_compressed(cv_v.at[pl.ds(ptra, L)], va - lo, mask=ma)
          plsc.store_compressed(cj_v.at[pl.ds(ptra, L)], ja, mask=ma)
          plsc.store_compressed(cv_v.at[pl.ds(SBASE + ptrb, L)], vb - lo,
                                mask=mb)
          plsc.store_compressed(cj_v.at[pl.ds(SBASE + ptrb, L)], jb, mask=mb)
          pa = plsc.all_reduce_population_count(ma)
          pb = plsc.all_reduce_population_count(mb)
          return ptra + pa[0], ptrb + pb[0]

        cnta, cntb = lax.fori_loop(0, SEG // (2 * L), scan_it,
                                   (jnp.int32(0), jnp.int32(0)), unroll=4)

        @pl.when(s == 0)
        def _():
          @pl.when(wid < 13)
          def _():
            cp_wide.wait()

          @pl.when(wid >= 13)
          def _():
            cp_narrow.wait()

          @pl.when(wid == 31)
          def _():
            cp_tail.wait()
            for r in range(32):
              for c2 in range(TAIL // L):
                stage_v[r, pl.ds(NARROW + c2 * L, L)] = (
                    tail_v[r, pl.ds(c2 * L, L)])

        def stream_chunks(base, cnt, carry):
          fired0, fired1, gc = carry
          nchunks = (cnt + CHUNK - 1) // CHUNK

          def per_chunk_pair(cp, carry):
            fired0, fired1, gc = carry
            c0 = cp * 2
            c1 = cp * 2 + 1

            @pl.when(c0 < nchunks)
            def _():
              @pl.when(fired0 == 1)
              def _():
                scat_copy(0).wait()
              fill_chunk(base, c0, cnt, 0)

            fired0 = jnp.where(c0 < nchunks, 1, fired0)

            @pl.when(c1 < nchunks)
            def _():
              @pl.when(fired1 == 1)
              def _():
                scat_copy(1).wait()
              fill_chunk(base, c1, cnt, 1)

            fired1 = jnp.where(c1 < nchunks, 1, fired1)
            return fired0, fired1, gc + jnp.where(c0 < nchunks, 1, 0) + \
                jnp.where(c1 < nchunks, 1, 0)

          npairs = (nchunks + 1) // 2
          return lax.fori_loop(0, npairs, per_chunk_pair,
                               (fired0, fired1, gc))

        carry = stream_chunks(0, cnta, (fired0, fired1, gc))
        return stream_chunks(SBASE, cntb, carry)

      return lax.fori_loop(0, NSEG, per_segment, (fired0, fired1, gc))

    fired0, fired1, _ = lax.fori_loop(
        0, NUM_FEATURES, per_feature,
        (jnp.int32(0), jnp.int32(0), jnp.int32(0)))

    @pl.when(fired0 == 1)
    def _():
      scat_copy(0).wait()

    @pl.when(fired1 == 1)
    def _():
      scat_copy(1).wait()

  return sc_gather


_sc_gather = _make_sc_gather()


@jax.jit
def kernel(features, tables):
  table_t = tables.transpose(0, 2, 1)
  feat_t = features.T
  out = _sc_gather(table_t, feat_t)
  out = out[:N, :EMBED_DIM]
  return out.reshape(BATCH, NUM_FEATURES * EMBED_DIM)


# speedup vs baseline: 1.3898x; 1.3898x over previous
import functools

import jax
import jax.numpy as jnp
from jax import lax
from jax.experimental import pallas as pl
from jax.experimental.pallas import tpu as pltpu
from jax.experimental.pallas import tpu_sc as plsc

NUM_FEATURES = 26
VOCAB = 100000
EMBED_DIM = 32
BATCH = 16384
N = BATCH * NUM_FEATURES
L = 16

SEG = 2048
NSEG = BATCH // SEG
WIDE = 3200
NARROW = 3072
TAIL = 32
SPLIT = 13 * WIDE
CHUNK = 32
GPC = CHUNK // L


def _make_sc_gather():
  mesh = plsc.VectorSubcoreMesh(core_axis_name="c", subcore_axis_name="s")

  @functools.partial(
      pl.kernel,
      mesh=mesh,
      compiler_params=pltpu.CompilerParams(
          use_tc_tiling_on_sc=True, needs_layout_passes=False),
      out_type=jax.ShapeDtypeStruct((N + CHUNK, 128), jnp.float32),
      scratch_types=[
          pltpu.VMEM((32, WIDE), jnp.float32),
          pltpu.VMEM((SEG,), jnp.int32),
          pltpu.VMEM((SEG,), jnp.int32),
          pltpu.VMEM((SEG + L,), jnp.int32),
          pltpu.VMEM((SEG + L,), jnp.int32),
          pltpu.VMEM((CHUNK, 128), jnp.float32),
          pltpu.VMEM((CHUNK, 128), jnp.float32),
          pltpu.VMEM((CHUNK,), jnp.int32),
          pltpu.VMEM((CHUNK,), jnp.int32),
          pltpu.VMEM((32, TAIL), jnp.float32),
          pltpu.SemaphoreType.DMA,
          pltpu.SemaphoreType.DMA,
          pltpu.SemaphoreType.DMA,
          pltpu.SemaphoreType.DMA,
          pltpu.SemaphoreType.DMA,
      ],
  )
  def sc_gather(table_hbm, feat_hbm, out_hbm, stage_v, fs0_v, fs1_v, cv_v,
                cj_v, outst0_v, outst1_v, sidx0_v, sidx1_v, tail_v, sem_stage,
                sem_s0, sem_s1, sem_f0, sem_f1):
    info = plsc.get_sparse_core_info()
    nc = info.num_cores
    wid = lax.axis_index("s") * nc + lax.axis_index("c")

    lo = jnp.where(wid < 13, wid * WIDE, SPLIT + (wid - 13) * NARROW)
    hi = lo + jnp.where(wid < 13, WIDE, NARROW) + jnp.where(
        wid == 31, TAIL, 0)

    iota = lax.iota(jnp.int32, L)
    iota26 = iota * NUM_FEATURES

    outst = (outst0_v, outst1_v)
    sidx = (sidx0_v, sidx1_v)
    sems = (sem_s0, sem_s1)

    def scat_copy(p):
      return pltpu.make_async_copy(outst[p], out_hbm.at[sidx[p]], sems[p])

    def fill_chunk(c, cnt, p):
      for g in range(GPC):
        off = c * CHUNK + g * L
        valid = (off + iota) < cnt
        vloc = cv_v[pl.ds(off, L)]
        vloc = jnp.where(valid, vloc, 0)
        j = cj_v[pl.ds(off, L)]
        j = jnp.where(valid, j, N + g * L + iota)
        sidx[p][pl.ds(g * L, L)] = j
        rows = lax.iota(jnp.int32, L) + g * L
        for e0 in range(0, EMBED_DIM, 8):
          vals = [
              plsc.load_gather(
                  stage_v, [jnp.full((L,), e0 + i, jnp.int32), vloc])
              for i in range(8)
          ]
          for i in range(8):
            plsc.store_scatter(
                outst[p], [rows, jnp.full((L,), e0 + i, jnp.int32)], vals[i])
      scat_copy(p).start()

    def per_feature(f, carry):

      cp_wide = pltpu.make_async_copy(
          table_hbm.at[f, :, pl.ds(lo, WIDE)], stage_v, sem_stage)
      cp_narrow = pltpu.make_async_copy(
          table_hbm.at[f, :, pl.ds(lo, NARROW)],
          stage_v.at[:, pl.ds(0, NARROW)], sem_stage)
      cp_tail = pltpu.make_async_copy(
          table_hbm.at[f, :, pl.ds(VOCAB - TAIL, TAIL)], tail_v, sem_stage)

      @pl.when(wid < 13)
      def _():
        cp_wide.start()

      @pl.when(wid >= 13)
      def _():
        cp_narrow.start()

      @pl.when(wid == 31)
      def _():
        cp_tail.start()

      fs = (fs0_v, fs1_v)
      semf = (sem_f0, sem_f1)

      def feat_copy(s, p):
        return pltpu.make_async_copy(
            feat_hbm.at[f, pl.ds(s * SEG, SEG)], fs[p], semf[p])

      feat_copy(0, 0).start()

      carry = carry
      for s in range(NSEG):
        fired0, fired1, gc = carry
        p_seg = s % 2
        featseg_v = fs[p_seg]
        feat_copy(s, p_seg).wait()
        if s + 1 < NSEG:
          feat_copy(s + 1, 1 - p_seg).start()

        def scan_it(k, ptr, featseg_v=featseg_v, s=s):
          v = featseg_v[pl.ds(k * L, L)]
          m = (v >= lo) & (v < hi)
          vloc = v - lo
          j = iota26 + ((s * SEG + k * L) * NUM_FEATURES + f)
          plsc.store_compressed(cv_v.at[pl.ds(ptr, L)], vloc, mask=m)
          plsc.store_compressed(cj_v.at[pl.ds(ptr, L)], j, mask=m)
          pc = plsc.all_reduce_population_count(m)
          return ptr + pc[0]

        cnt = lax.fori_loop(0, SEG // L, scan_it, jnp.int32(0), unroll=8)

        if s == 0:
          @pl.when(wid < 13)
          def _():
            cp_wide.wait()

          @pl.when(wid >= 13)
          def _():
            cp_narrow.wait()

          @pl.when(wid == 31)
          def _():
            cp_tail.wait()
            for r in range(32):
              for c2 in range(TAIL // L):
                stage_v[r, pl.ds(NARROW + c2 * L, L)] = (
                    tail_v[r, pl.ds(c2 * L, L)])

        nchunks = (cnt + CHUNK - 1) // CHUNK

        def per_chunk_pair(cp, carry, cnt=cnt, nchunks=nchunks):
          fired0, fired1, gc = carry
          c0 = cp * 2
          c1 = cp * 2 + 1

          @pl.when(c0 < nchunks)
          def _():
            @pl.when(fired0 == 1)
            def _():
              scat_copy(0).wait()
            fill_chunk(c0, cnt, 0)

          fired0 = jnp.where(c0 < nchunks, 1, fired0)

          @pl.when(c1 < nchunks)
          def _():
            @pl.when(fired1 == 1)
            def _():
              scat_copy(1).wait()
            fill_chunk(c1, cnt, 1)

          fired1 = jnp.where(c1 < nchunks, 1, fired1)
          return fired0, fired1, gc + jnp.where(c0 < nchunks, 1, 0) + \
              jnp.where(c1 < nchunks, 1, 0)

        npairs = (nchunks + 1) // 2
        carry = lax.fori_loop(0, npairs, per_chunk_pair,
                              (fired0, fired1, gc))
      return carry

    fired0, fired1, _ = lax.fori_loop(
        0, NUM_FEATURES, per_feature,
        (jnp.int32(0), jnp.int32(0), jnp.int32(0)))

    @pl.when(fired0 == 1)
    def _():
      scat_copy(0).wait()

    @pl.when(fired1 == 1)
    def _():
      scat_copy(1).wait()

  return sc_gather


_sc_gather = _make_sc_gather()


@jax.jit
def kernel(features, tables):
  table_t = tables.transpose(0, 2, 1)
  feat_t = features.T
  out = _sc_gather(table_t, feat_t)
  out = out[:N, :EMBED_DIM]
  return out.reshape(BATCH, NUM_FEATURES * EMBED_DIM)


# speedup vs baseline: 1.5289x; 1.1001x over previous
import functools

import jax
import jax.numpy as jnp
from jax import lax
from jax.experimental import pallas as pl
from jax.experimental.pallas import tpu as pltpu
from jax.experimental.pallas import tpu_sc as plsc

NUM_FEATURES = 26
VOCAB = 100000
EMBED_DIM = 32
BATCH = 16384
N = BATCH * NUM_FEATURES
L = 16

SEG = 2048
NSEG = BATCH // SEG
WIDE = 3200
NARROW = 3072
TAIL = 32
SPLIT = 13 * WIDE
CHUNK = 32
GPC = CHUNK // L


def _make_sc_gather():
  mesh = plsc.VectorSubcoreMesh(core_axis_name="c", subcore_axis_name="s")

  @functools.partial(
      pl.kernel,
      mesh=mesh,
      compiler_params=pltpu.CompilerParams(
          use_tc_tiling_on_sc=True, needs_layout_passes=False),
      out_type=jax.ShapeDtypeStruct((N + CHUNK, 128), jnp.float32),
      scratch_types=[
          pltpu.VMEM((32, WIDE), jnp.float32),
          pltpu.VMEM((SEG,), jnp.int32),
          pltpu.VMEM((SEG,), jnp.int32),
          pltpu.VMEM((SEG + CHUNK + 2 * L,), jnp.int32),
          pltpu.VMEM((SEG + CHUNK + 2 * L,), jnp.int32),
          pltpu.VMEM((CHUNK, 128), jnp.float32),
          pltpu.VMEM((CHUNK, 128), jnp.float32),
          pltpu.VMEM((CHUNK,), jnp.int32),
          pltpu.VMEM((CHUNK,), jnp.int32),
          pltpu.VMEM((32, TAIL), jnp.float32),
          pltpu.SemaphoreType.DMA,
          pltpu.SemaphoreType.DMA,
          pltpu.SemaphoreType.DMA,
          pltpu.SemaphoreType.DMA,
          pltpu.SemaphoreType.DMA,
      ],
  )
  def sc_gather(table_hbm, feat_hbm, out_hbm, stage_v, fs0_v, fs1_v, cv_v,
                cj_v, outst0_v, outst1_v, sidx0_v, sidx1_v, tail_v, sem_stage,
                sem_s0, sem_s1, sem_f0, sem_f1):
    info = plsc.get_sparse_core_info()
    nc = info.num_cores
    wid = lax.axis_index("s") * nc + lax.axis_index("c")

    lo = jnp.where(wid < 13, wid * WIDE, SPLIT + (wid - 13) * NARROW)
    hi = lo + jnp.where(wid < 13, WIDE, NARROW) + jnp.where(
        wid == 31, TAIL, 0)

    iota = lax.iota(jnp.int32, L)
    iota26 = iota * NUM_FEATURES

    outst = (outst0_v, outst1_v)
    sidx = (sidx0_v, sidx1_v)
    sems = (sem_s0, sem_s1)

    def scat_copy(p):
      return pltpu.make_async_copy(outst[p], out_hbm.at[sidx[p]], sems[p])

    def fill_chunk(start, cnt, p):
      for g in range(GPC):
        off = start + g * L
        valid = (off + iota) < cnt
        vloc = cv_v[pl.ds(off, L)]
        vloc = jnp.where(valid, vloc, 0)
        j = cj_v[pl.ds(off, L)]
        j = jnp.where(valid, j, N + g * L + iota)
        sidx[p][pl.ds(g * L, L)] = j
        rows = lax.iota(jnp.int32, L) + g * L
        for e0 in range(0, EMBED_DIM, 8):
          vals = [
              plsc.load_gather(
                  stage_v, [jnp.full((L,), e0 + i, jnp.int32), vloc])
              for i in range(8)
          ]
          for i in range(8):
            plsc.store_scatter(
                outst[p], [rows, jnp.full((L,), e0 + i, jnp.int32)], vals[i])
      scat_copy(p).start()

    def per_feature(f, carry):

      cp_wide = pltpu.make_async_copy(
          table_hbm.at[f, :, pl.ds(lo, WIDE)], stage_v, sem_stage)
      cp_narrow = pltpu.make_async_copy(
          table_hbm.at[f, :, pl.ds(lo, NARROW)],
          stage_v.at[:, pl.ds(0, NARROW)], sem_stage)
      cp_tail = pltpu.make_async_copy(
          table_hbm.at[f, :, pl.ds(VOCAB - TAIL, TAIL)], tail_v, sem_stage)

      @pl.when(wid < 13)
      def _():
        cp_wide.start()

      @pl.when(wid >= 13)
      def _():
        cp_narrow.start()

      @pl.when(wid == 31)
      def _():
        cp_tail.start()

      fs = (fs0_v, fs1_v)
      semf = (sem_f0, sem_f1)

      def feat_copy(s, p):
        return pltpu.make_async_copy(
            feat_hbm.at[f, pl.ds(s * SEG, SEG)], fs[p], semf[p])

      feat_copy(0, 0).start()

      base = jnp.int32(0)
      cnt = jnp.int32(0)
      for s in range(NSEG):
        fired0, fired1, gc = carry
        p_seg = s % 2
        featseg_v = fs[p_seg]
        feat_copy(s, p_seg).wait()
        if s + 1 < NSEG:
          feat_copy(s + 1, 1 - p_seg).start()

        def scan_it(k, ptr, featseg_v=featseg_v, s=s):
          v = featseg_v[pl.ds(k * L, L)]
          m = (v >= lo) & (v < hi)
          vloc = v - lo
          j = iota26 + ((s * SEG + k * L) * NUM_FEATURES + f)
          plsc.store_compressed(cv_v.at[pl.ds(ptr, L)], vloc, mask=m)
          plsc.store_compressed(cj_v.at[pl.ds(ptr, L)], j, mask=m)
          pc = plsc.all_reduce_population_count(m)
          return ptr + pc[0]

        cnt = lax.fori_loop(0, SEG // L, scan_it, cnt, unroll=8)

        if s == 0:
          @pl.when(wid < 13)
          def _():
            cp_wide.wait()

          @pl.when(wid >= 13)
          def _():
            cp_narrow.wait()

          @pl.when(wid == 31)
          def _():
            cp_tail.wait()
            for r in range(32):
              for c2 in range(TAIL // L):
                stage_v[r, pl.ds(NARROW + c2 * L, L)] = (
                    tail_v[r, pl.ds(c2 * L, L)])

        nfull = (cnt - base) // CHUNK

        def per_chunk_pair(cp, carry, cnt=cnt, nfull=nfull, base=base):
          fired0, fired1, gc = carry
          c0 = cp * 2
          c1 = cp * 2 + 1

          @pl.when(c0 < nfull)
          def _():
            @pl.when(fired0 == 1)
            def _():
              scat_copy(0).wait()
            fill_chunk(base + c0 * CHUNK, cnt, 0)

          fired0 = jnp.where(c0 < nfull, 1, fired0)

          @pl.when(c1 < nfull)
          def _():
            @pl.when(fired1 == 1)
            def _():
              scat_copy(1).wait()
            fill_chunk(base + c1 * CHUNK, cnt, 1)

          fired1 = jnp.where(c1 < nfull, 1, fired1)
          return fired0, fired1, gc + jnp.where(c0 < nfull, 1, 0) + \
              jnp.where(c1 < nfull, 1, 0)

        npairs = (nfull + 1) // 2
        carry = lax.fori_loop(0, npairs, per_chunk_pair,
                              (fired0, fired1, gc))
        base = base + nfull * CHUNK
        for w2 in range(CHUNK // L):
          cv_v[pl.ds(w2 * L, L)] = cv_v[pl.ds(base + w2 * L, L)]
          cj_v[pl.ds(w2 * L, L)] = cj_v[pl.ds(base + w2 * L, L)]
        cnt = cnt - base
        base = jnp.int32(0)

      fired0, fired1, gc = carry

      @pl.when(cnt > base)
      def _():
        @pl.when(fired0 == 1)
        def _():
          scat_copy(0).wait()
        fill_chunk(base, cnt, 0)

      fired0 = jnp.where(cnt > base, 1, fired0)
      return fired0, fired1, gc

    fired0, fired1, _ = lax.fori_loop(
        0, NUM_FEATURES, per_feature,
        (jnp.int32(0), jnp.int32(0), jnp.int32(0)))

    @pl.when(fired0 == 1)
    def _():
      scat_copy(0).wait()

    @pl.when(fired1 == 1)
    def _():
      scat_copy(1).wait()

  return sc_gather


_sc_gather = _make_sc_gather()


@jax.jit
def kernel(features, tables):
  table_t = tables.transpose(0, 2, 1)
  feat_t = features.T
  out = _sc_gather(table_t, feat_t)
  out = out[:N, :EMBED_DIM]
  return out.reshape(BATCH, NUM_FEATURES * EMBED_DIM)
